# baseline (speedup 1.0000x reference)
import functools

import jax
import jax.numpy as jnp
from jax import lax
from jax.experimental import pallas as pl
from jax.experimental.pallas import tpu as pltpu
from jax.experimental.pallas import tpu_sc as plsc

BATCH = 16384
EMBED_DIM = 64
NC = 2
NS = 16
NW = NC * NS
ROWS_PER_W = BATCH // NW
CHUNK = 128
NCHUNKS = ROWS_PER_W // CHUNK
LANES = 16
DCHUNKS = EMBED_DIM // LANES


def _sc_similarity(uidx2d, midx2d, user_table, movie_table):
    mesh = plsc.VectorSubcoreMesh(core_axis_name="c", subcore_axis_name="s")

    @functools.partial(
        pl.kernel,
        mesh=mesh,
        compiler_params=pltpu.CompilerParams(use_tc_tiling_on_sc=False),
        out_type=jax.ShapeDtypeStruct((BATCH,), jnp.float32),
        scratch_types=[
            pltpu.VMEM((NCHUNKS, CHUNK), jnp.int32),
            pltpu.VMEM((NCHUNKS, CHUNK), jnp.int32),
            pltpu.VMEM((ROWS_PER_W, EMBED_DIM), jnp.float32),
            pltpu.VMEM((ROWS_PER_W, EMBED_DIM), jnp.float32),
            pltpu.VMEM((2 * LANES,), jnp.float32),
            pltpu.VMEM((ROWS_PER_W,), jnp.float32),
            pltpu.SemaphoreType.DMA,
        ],
    )
    def k(uidx_hbm, midx_hbm, utab_hbm, mtab_hbm, out_hbm,
          uidx_v, midx_v, urows_v, mrows_v, scr, out_v, sem):
        wid = lax.axis_index("s") * NC + lax.axis_index("c")
        ibase = wid * NCHUNKS

        pltpu.sync_copy(uidx_hbm.at[pl.ds(ibase, NCHUNKS)], uidx_v)
        pltpu.sync_copy(midx_hbm.at[pl.ds(ibase, NCHUNKS)], midx_v)

        copies = []
        for g in range(NCHUNKS):
            copies.append(pltpu.async_copy(
                utab_hbm.at[uidx_v.at[g]],
                urows_v.at[pl.ds(g * CHUNK, CHUNK)], sem))
            copies.append(pltpu.async_copy(
                mtab_hbm.at[midx_v.at[g]],
                mrows_v.at[pl.ds(g * CHUNK, CHUNK)], sem))
        for c in copies:
            c.wait()

        lane_iota = lax.iota(jnp.int32, LANES)

        def group_body(g, carry):
            base = g * LANES
            tot = jnp.zeros((LANES,), jnp.float32)
            for r in range(LANES):
                row = base + r
                acc = (urows_v[row, pl.ds(0, LANES)]
                       * mrows_v[row, pl.ds(0, LANES)])
                for c in range(1, DCHUNKS):
                    acc = acc + (urows_v[row, pl.ds(c * LANES, LANES)]
                                 * mrows_v[row, pl.ds(c * LANES, LANES)])
                for k in (8, 4, 2, 1):
                    scr[pl.ds(0, LANES)] = acc
                    scr[pl.ds(LANES, LANES)] = acc
                    acc = acc + scr[pl.ds(k, LANES)]
                tot = jnp.where(lane_iota == r, acc, tot)
            out_v[pl.ds(base, LANES)] = tot
            return carry

        lax.fori_loop(0, ROWS_PER_W // LANES, group_body, 0)

        pltpu.sync_copy(out_v, out_hbm.at[pl.ds(wid * ROWS_PER_W, ROWS_PER_W)])

    return k(uidx2d, midx2d, user_table, movie_table)


def kernel(user_ids, movie_ids, side_info_ids, user_table, movie_table,
           side_table, alpha):
    del side_info_ids, side_table, alpha
    uidx = user_ids.astype(jnp.int32).reshape(BATCH // CHUNK, CHUNK)
    midx = movie_ids.astype(jnp.int32).reshape(BATCH // CHUNK, CHUNK)
    return _sc_similarity(uidx, midx, user_table, movie_table)
